# baseline (speedup 1.0000x reference)
import functools

import jax
import jax.numpy as jnp
from jax import lax
from jax.experimental import pallas as pl
from jax.experimental.pallas import tpu as pltpu
from jax.experimental.pallas import tpu_sc as plsc

B, S, D = 4, 2048, 1024
NC, NS = 2, 16
NW = NC * NS
S_PER_W = S // NW
CH = 16
N_CH = S_PER_W // CH
CHW = CH * D
LANES = 16


def _sc_body(x_hbm, emb_hbm, out_hbm, emb_v, x_v):
    wid = lax.axis_index("s") * NC + lax.axis_index("c")
    base = wid * (S_PER_W * D)

    def add_chunk(i, _):
        off = i * LANES
        x_v[pl.ds(off, LANES)] = x_v[pl.ds(off, LANES)] + emb_v[pl.ds(off, LANES)]
        return 0

    for ci in range(N_CH):
        e_off = base + ci * CHW
        pltpu.sync_copy(emb_hbm.at[pl.ds(e_off, CHW)], emb_v)
        for b in range(B):
            x_off = b * (S * D) + e_off
            pltpu.sync_copy(x_hbm.at[pl.ds(x_off, CHW)], x_v)
            lax.fori_loop(0, CHW // LANES, add_chunk, 0)
            pltpu.sync_copy(x_v, out_hbm.at[pl.ds(x_off, CHW)])


@functools.cache
def _make_sc_add():
    return pl.kernel(
        _sc_body,
        out_type=jax.ShapeDtypeStruct((B * S * D,), jnp.float32),
        mesh=plsc.VectorSubcoreMesh(
            core_axis_name="c", subcore_axis_name="s", num_cores=NC, num_subcores=NS
        ),
        scratch_types=[
            pltpu.VMEM((CHW,), jnp.float32),
            pltpu.VMEM((CHW,), jnp.float32),
        ],
    )


def kernel(x, emb):
    out = _make_sc_add()(x.reshape(-1), emb.reshape(-1))
    return out.reshape(B, S, D)


# speedup vs baseline: 1.5607x; 1.5607x over previous
import functools

import jax
import jax.numpy as jnp
from jax import lax
from jax.experimental import pallas as pl
from jax.experimental.pallas import tpu as pltpu
from jax.experimental.pallas import tpu_sc as plsc

B, S, D = 4, 2048, 1024
NC, NS = 2, 16
NW = NC * NS
S_PER_W = S // NW
CH = 8
N_CH = S_PER_W // CH
CHW = CH * D
LANES = 16


def _sc_body(x_hbm, emb_hbm, out_hbm, emb_v, x_v, e_sem, x_sem, o_sem):
    wid = lax.axis_index("s") * NC + lax.axis_index("c")
    base = wid * (S_PER_W * D)

    def issue_in(ci, p):
        e_off = base + ci * CHW
        ed = pltpu.async_copy(
            emb_hbm.at[pl.ds(e_off, CHW)], emb_v.at[p], e_sem.at[p]
        )
        xds = [
            pltpu.async_copy(
                x_hbm.at[pl.ds(b * (S * D) + e_off, CHW)],
                x_v.at[p, b],
                x_sem.at[p],
            )
            for b in range(B)
        ]
        return ed, xds

    def issue_out(ci, p):
        e_off = base + ci * CHW
        return [
            pltpu.async_copy(
                x_v.at[p, b],
                out_hbm.at[pl.ds(b * (S * D) + e_off, CHW)],
                o_sem.at[p],
            )
            for b in range(B)
        ]

    pend_in = {0: issue_in(0, 0)}
    pend_out = {}
    for ci in range(N_CH):
        p = ci % 2
        if ci + 1 < N_CH:
            if ci - 1 in pend_out:
                for od in pend_out.pop(ci - 1):
                    od.wait()
            pend_in[ci + 1] = issue_in(ci + 1, (ci + 1) % 2)
        ed, xds = pend_in.pop(ci)
        ed.wait()
        for xd in xds:
            xd.wait()

        @plsc.parallel_loop(0, CHW // LANES, unroll=4)
        def _(j):
            sl = pl.ds(j * LANES, LANES)
            ev = emb_v[p, sl]
            for b in range(B):
                x_v[p, b, sl] = x_v[p, b, sl] + ev

        pend_out[ci] = issue_out(ci, p)
    for ods in pend_out.values():
        for od in ods:
            od.wait()


@functools.cache
def _make_sc_add():
    return pl.kernel(
        _sc_body,
        out_type=jax.ShapeDtypeStruct((B * S * D,), jnp.float32),
        mesh=plsc.VectorSubcoreMesh(
            core_axis_name="c", subcore_axis_name="s", num_cores=NC, num_subcores=NS
        ),
        scratch_types=[
            pltpu.VMEM((2, CHW), jnp.float32),
            pltpu.VMEM((2, B, CHW), jnp.float32),
            pltpu.SemaphoreType.DMA((2,)),
            pltpu.SemaphoreType.DMA((2,)),
            pltpu.SemaphoreType.DMA((2,)),
        ],
    )


def kernel(x, emb):
    out = _make_sc_add()(x.reshape(-1), emb.reshape(-1))
    return out.reshape(B, S, D)


# speedup vs baseline: 4.1429x; 2.6544x over previous
import functools

import jax
import jax.numpy as jnp
from jax import lax
from jax.experimental import pallas as pl
from jax.experimental.pallas import tpu as pltpu
from jax.experimental.pallas import tpu_sc as plsc

B, S, D = 4, 2048, 1024
NC, NS = 2, 16
NW = NC * NS
S_PER_W = S // NW
CH = 8
N_CH = S_PER_W // CH
LANES = 16
VPR = D // LANES


def _sc_body(x_hbm, emb_hbm, out_hbm, emb_v, x_v, e_sem, x_sem, o_sem):
    wid = lax.axis_index("s") * NC + lax.axis_index("c")
    s0 = wid * S_PER_W

    def issue_in(ci, p):
        r = s0 + ci * CH
        ed = pltpu.async_copy(
            emb_hbm.at[pl.ds(r, CH)], emb_v.at[p], e_sem.at[p]
        )
        xds = [
            pltpu.async_copy(
                x_hbm.at[pl.ds(b * S + r, CH)], x_v.at[p, b], x_sem.at[p]
            )
            for b in range(B)
        ]
        return ed, xds

    def issue_out(ci, p):
        r = s0 + ci * CH
        return [
            pltpu.async_copy(
                x_v.at[p, b], out_hbm.at[pl.ds(b * S + r, CH)], o_sem.at[p]
            )
            for b in range(B)
        ]

    pend_in = {0: issue_in(0, 0)}
    pend_out = {}
    for ci in range(N_CH):
        p = ci % 2
        if ci + 1 < N_CH:
            if ci - 1 in pend_out:
                for od in pend_out.pop(ci - 1):
                    od.wait()
            pend_in[ci + 1] = issue_in(ci + 1, (ci + 1) % 2)
        ed, xds = pend_in.pop(ci)
        ed.wait()
        for xd in xds:
            xd.wait()

        @plsc.parallel_loop(0, CH * VPR, unroll=4)
        def _(j):
            r = lax.shift_right_logical(j, 6)
            c = lax.bitwise_and(j, VPR - 1)
            sl = pl.ds(c * LANES, LANES)
            ev = emb_v[p, r, sl]
            for b in range(B):
                x_v[p, b, r, sl] = x_v[p, b, r, sl] + ev

        pend_out[ci] = issue_out(ci, p)
    for ods in pend_out.values():
        for od in ods:
            od.wait()


@functools.cache
def _make_sc_add():
    return pl.kernel(
        _sc_body,
        out_type=jax.ShapeDtypeStruct((B * S, D), jnp.float32),
        mesh=plsc.VectorSubcoreMesh(
            core_axis_name="c", subcore_axis_name="s", num_cores=NC, num_subcores=NS
        ),
        scratch_types=[
            pltpu.VMEM((2, CH, D), jnp.float32),
            pltpu.VMEM((2, B, CH, D), jnp.float32),
            pltpu.SemaphoreType.DMA((2,)),
            pltpu.SemaphoreType.DMA((2,)),
            pltpu.SemaphoreType.DMA((2,)),
        ],
    )


def kernel(x, emb):
    out = _make_sc_add()(x.reshape(B * S, D), emb)
    return out.reshape(B, S, D)


# speedup vs baseline: 4.2208x; 1.0188x over previous
import functools

import jax
import jax.numpy as jnp
from jax import lax
from jax.experimental import pallas as pl
from jax.experimental.pallas import tpu as pltpu
from jax.experimental.pallas import tpu_sc as plsc

B, S, D = 4, 2048, 1024
NC, NS = 2, 16
NW = NC * NS
S_PER_W = S // NW
CH = 8
N_CH = S_PER_W // CH
LANES = 16
VPR = D // LANES
NBUF = 3


def _sc_body(x_hbm, emb_hbm, out_hbm, emb_v, x_v, e_sem, x_sem, o_sem):
    wid = lax.axis_index("s") * NC + lax.axis_index("c")
    s0 = wid * S_PER_W

    def issue_in(ci, p):
        r = s0 + ci * CH
        ed = pltpu.async_copy(
            emb_hbm.at[pl.ds(r, CH)], emb_v.at[p], e_sem.at[p]
        )
        xds = [
            pltpu.async_copy(
                x_hbm.at[pl.ds(b * S + r, CH)], x_v.at[p, b], x_sem.at[p]
            )
            for b in range(B)
        ]
        return ed, xds

    def issue_out(ci, p):
        r = s0 + ci * CH
        return [
            pltpu.async_copy(
                x_v.at[p, b], out_hbm.at[pl.ds(b * S + r, CH)], o_sem.at[p]
            )
            for b in range(B)
        ]

    pend_in = {0: issue_in(0, 0), 1: issue_in(1, 1)}
    pend_out = {}
    for ci in range(N_CH):
        p = ci % NBUF
        ed, xds = pend_in.pop(ci)
        ed.wait()
        for xd in xds:
            xd.wait()

        @plsc.parallel_loop(0, CH * VPR, unroll=4)
        def _(j):
            r = lax.shift_right_logical(j, 6)
            c = lax.bitwise_and(j, VPR - 1)
            sl = pl.ds(c * LANES, LANES)
            ev = emb_v[p, r, sl]
            for b in range(B):
                x_v[p, b, r, sl] = x_v[p, b, r, sl] + ev

        pend_out[ci] = issue_out(ci, p)
        if ci + 2 < N_CH:
            if ci - 1 in pend_out:
                for od in pend_out.pop(ci - 1):
                    od.wait()
            pend_in[ci + 2] = issue_in(ci + 2, (ci + 2) % NBUF)
    for ods in pend_out.values():
        for od in ods:
            od.wait()


@functools.cache
def _make_sc_add():
    return pl.kernel(
        _sc_body,
        out_type=jax.ShapeDtypeStruct((B * S, D), jnp.float32),
        mesh=plsc.VectorSubcoreMesh(
            core_axis_name="c", subcore_axis_name="s", num_cores=NC, num_subcores=NS
        ),
        scratch_types=[
            pltpu.VMEM((NBUF, CH, D), jnp.float32),
            pltpu.VMEM((NBUF, B, CH, D), jnp.float32),
            pltpu.SemaphoreType.DMA((NBUF,)),
            pltpu.SemaphoreType.DMA((NBUF,)),
            pltpu.SemaphoreType.DMA((NBUF,)),
        ],
    )


def kernel(x, emb):
    out = _make_sc_add()(x.reshape(B * S, D), emb)
    return out.reshape(B, S, D)


# speedup vs baseline: 4.2694x; 1.0115x over previous
import functools

import jax
import jax.numpy as jnp
from jax import lax
from jax.experimental import pallas as pl
from jax.experimental.pallas import tpu as pltpu
from jax.experimental.pallas import tpu_sc as plsc

B, S, D = 4, 2048, 1024
NC, NS = 2, 16
NW = NC * NS
S_PER_W = S // NW
CH = 8
N_CH = S_PER_W // CH
LANES = 16
VPR = D // LANES
NBUF = 3


def _sc_body(x_hbm, emb_hbm, out_hbm, emb_v, x_v, e_sem, x_sem, o_sem):
    wid = lax.axis_index("s") * NC + lax.axis_index("c")
    s0 = wid * S_PER_W

    def issue_in(ci, p):
        r = s0 + ci * CH
        ed = pltpu.async_copy(
            emb_hbm.at[pl.ds(r, CH)], emb_v.at[p], e_sem.at[p]
        )
        xds = [
            pltpu.async_copy(
                x_hbm.at[:, pl.ds(r, CH), :], x_v.at[p], x_sem.at[p]
            )
        ]
        return ed, xds

    def issue_out(ci, p):
        r = s0 + ci * CH
        return [
            pltpu.async_copy(
                x_v.at[p], out_hbm.at[:, pl.ds(r, CH), :], o_sem.at[p]
            )
        ]

    pend_in = {0: issue_in(0, 0), 1: issue_in(1, 1)}
    pend_out = {}
    for ci in range(N_CH):
        p = ci % NBUF
        ed, xds = pend_in.pop(ci)
        ed.wait()
        for xd in xds:
            xd.wait()

        @plsc.parallel_loop(0, CH * VPR, unroll=4)
        def _(j):
            r = lax.shift_right_logical(j, 6)
            c = lax.bitwise_and(j, VPR - 1)
            sl = pl.ds(c * LANES, LANES)
            ev = emb_v[p, r, sl]
            for b in range(B):
                x_v[p, b, r, sl] = x_v[p, b, r, sl] + ev

        pend_out[ci] = issue_out(ci, p)
        if ci + 2 < N_CH:
            if ci - 1 in pend_out:
                for od in pend_out.pop(ci - 1):
                    od.wait()
            pend_in[ci + 2] = issue_in(ci + 2, (ci + 2) % NBUF)
    for ods in pend_out.values():
        for od in ods:
            od.wait()


@functools.cache
def _make_sc_add():
    return pl.kernel(
        _sc_body,
        out_type=jax.ShapeDtypeStruct((B, S, D), jnp.float32),
        mesh=plsc.VectorSubcoreMesh(
            core_axis_name="c", subcore_axis_name="s", num_cores=NC, num_subcores=NS
        ),
        scratch_types=[
            pltpu.VMEM((NBUF, CH, D), jnp.float32),
            pltpu.VMEM((NBUF, B, CH, D), jnp.float32),
            pltpu.SemaphoreType.DMA((NBUF,)),
            pltpu.SemaphoreType.DMA((NBUF,)),
            pltpu.SemaphoreType.DMA((NBUF,)),
        ],
    )


def kernel(x, emb):
    return _make_sc_add()(x, emb)


# speedup vs baseline: 4.3130x; 1.0102x over previous
import functools

import jax
import jax.numpy as jnp
from jax import lax
from jax.experimental import pallas as pl
from jax.experimental.pallas import tpu as pltpu
from jax.experimental.pallas import tpu_sc as plsc

B, S, D = 4, 2048, 1024
LANES = 16
VPR = D // LANES

S_TC = 1792
S_SC = S - S_TC
BS_TC = 448

NC, NS = 2, 16
NW = NC * NS
S_PER_W = S_SC // NW
CH = 8
N_CH = S_PER_W // CH
NBUF = 3


def _sc_body(x_hbm, emb_hbm, out_hbm, emb_v, x_v, e_sem, x_sem, o_sem):
    wid = lax.axis_index("s") * NC + lax.axis_index("c")
    s0 = wid * S_PER_W

    def issue_in(ci, p):
        r = s0 + ci * CH
        ed = pltpu.async_copy(
            emb_hbm.at[pl.ds(S_TC + r, CH)], emb_v.at[p], e_sem.at[p]
        )
        xd = pltpu.async_copy(
            x_hbm.at[:, pl.ds(S_TC + r, CH), :], x_v.at[p], x_sem.at[p]
        )
        return ed, xd

    def issue_out(ci, p):
        r = s0 + ci * CH
        return pltpu.async_copy(
            x_v.at[p], out_hbm.at[:, pl.ds(r, CH), :], o_sem.at[p]
        )

    pend_in = {ci: issue_in(ci, ci % NBUF) for ci in range(min(2, N_CH))}
    pend_out = {}
    for ci in range(N_CH):
        p = ci % NBUF
        ed, xd = pend_in.pop(ci)
        ed.wait()
        xd.wait()

        @plsc.parallel_loop(0, CH * VPR, unroll=4)
        def _(j):
            r = lax.shift_right_logical(j, 6)
            c = lax.bitwise_and(j, VPR - 1)
            sl = pl.ds(c * LANES, LANES)
            ev = emb_v[p, r, sl]
            for b in range(B):
                x_v[p, b, r, sl] = x_v[p, b, r, sl] + ev

        pend_out[ci] = issue_out(ci, p)
        if ci + 2 < N_CH:
            if ci - 1 in pend_out:
                pend_out.pop(ci - 1).wait()
            pend_in[ci + 2] = issue_in(ci + 2, (ci + 2) % NBUF)
    for od in pend_out.values():
        od.wait()


@functools.cache
def _make_sc_add():
    return pl.kernel(
        _sc_body,
        out_type=jax.ShapeDtypeStruct((B, S_SC, D), jnp.float32),
        mesh=plsc.VectorSubcoreMesh(
            core_axis_name="c", subcore_axis_name="s", num_cores=NC, num_subcores=NS
        ),
        scratch_types=[
            pltpu.VMEM((NBUF, CH, D), jnp.float32),
            pltpu.VMEM((NBUF, B, CH, D), jnp.float32),
            pltpu.SemaphoreType.DMA((NBUF,)),
            pltpu.SemaphoreType.DMA((NBUF,)),
            pltpu.SemaphoreType.DMA((NBUF,)),
        ],
    )


def _tc_add_body(x_ref, emb_ref, o_ref):
    o_ref[...] = x_ref[...] + emb_ref[...][None]


def _tc_add(x, emb):
    grid = (S_TC // BS_TC, B)
    return pl.pallas_call(
        _tc_add_body,
        grid=grid,
        in_specs=[
            pl.BlockSpec((1, BS_TC, D), lambda i, j: (j, i, 0)),
            pl.BlockSpec((BS_TC, D), lambda i, j: (i, 0)),
        ],
        out_specs=pl.BlockSpec((1, BS_TC, D), lambda i, j: (j, i, 0)),
        out_shape=jax.ShapeDtypeStruct((B, S, D), x.dtype),
    )(x, emb)


def kernel(x, emb):
    sc_out = _make_sc_add()(x, emb)
    tc_out = _tc_add(x, emb)
    return lax.dynamic_update_slice(tc_out, sc_out, (0, S_TC, 0))


# speedup vs baseline: 4.6007x; 1.0667x over previous
import functools

import jax
import jax.numpy as jnp
from jax import lax
from jax.experimental import pallas as pl
from jax.experimental.pallas import tpu as pltpu
from jax.experimental.pallas import tpu_sc as plsc

B, S, D = 4, 2048, 1024
LANES = 16
VPR = D // LANES

S_TC = 1792
S_SC = S - S_TC
BS_TC = 1792

NC, NS = 2, 16
NW = NC * NS
S_PER_W = S_SC // NW
CH = 8
N_CH = S_PER_W // CH
NBUF = 3


def _sc_body(x_hbm, emb_hbm, out_hbm, emb_v, x_v, e_sem, x_sem, o_sem):
    wid = lax.axis_index("s") * NC + lax.axis_index("c")
    s0 = wid * S_PER_W

    def issue_in(ci, p):
        r = s0 + ci * CH
        ed = pltpu.async_copy(
            emb_hbm.at[pl.ds(S_TC + r, CH)], emb_v.at[p], e_sem.at[p]
        )
        xd = pltpu.async_copy(
            x_hbm.at[:, pl.ds(S_TC + r, CH), :], x_v.at[p], x_sem.at[p]
        )
        return ed, xd

    def issue_out(ci, p):
        r = s0 + ci * CH
        return pltpu.async_copy(
            x_v.at[p], out_hbm.at[:, pl.ds(r, CH), :], o_sem.at[p]
        )

    pend_in = {ci: issue_in(ci, ci % NBUF) for ci in range(min(2, N_CH))}
    pend_out = {}
    for ci in range(N_CH):
        p = ci % NBUF
        ed, xd = pend_in.pop(ci)
        ed.wait()
        xd.wait()

        @plsc.parallel_loop(0, CH * VPR, unroll=4)
        def _(j):
            r = lax.shift_right_logical(j, 6)
            c = lax.bitwise_and(j, VPR - 1)
            sl = pl.ds(c * LANES, LANES)
            ev = emb_v[p, r, sl]
            for b in range(B):
                x_v[p, b, r, sl] = x_v[p, b, r, sl] + ev

        pend_out[ci] = issue_out(ci, p)
        if ci + 2 < N_CH:
            if ci - 1 in pend_out:
                pend_out.pop(ci - 1).wait()
            pend_in[ci + 2] = issue_in(ci + 2, (ci + 2) % NBUF)
    for od in pend_out.values():
        od.wait()


@functools.cache
def _make_sc_add():
    return pl.kernel(
        _sc_body,
        out_type=jax.ShapeDtypeStruct((B, S_SC, D), jnp.float32),
        mesh=plsc.VectorSubcoreMesh(
            core_axis_name="c", subcore_axis_name="s", num_cores=NC, num_subcores=NS
        ),
        scratch_types=[
            pltpu.VMEM((NBUF, CH, D), jnp.float32),
            pltpu.VMEM((NBUF, B, CH, D), jnp.float32),
            pltpu.SemaphoreType.DMA((NBUF,)),
            pltpu.SemaphoreType.DMA((NBUF,)),
            pltpu.SemaphoreType.DMA((NBUF,)),
        ],
    )


def _tc_add_body(x_ref, emb_ref, o_ref):
    o_ref[...] = x_ref[...] + emb_ref[...][None]


def _tc_add(x, emb):
    grid = (S_TC // BS_TC, B)
    return pl.pallas_call(
        _tc_add_body,
        grid=grid,
        in_specs=[
            pl.BlockSpec((1, BS_TC, D), lambda i, j: (j, i, 0)),
            pl.BlockSpec((BS_TC, D), lambda i, j: (i, 0)),
        ],
        out_specs=pl.BlockSpec((1, BS_TC, D), lambda i, j: (j, i, 0)),
        out_shape=jax.ShapeDtypeStruct((B, S, D), x.dtype),
    )(x, emb)


def _stitch_body(tc_ref, sc_ref, o_ref):
    o_ref[...] = sc_ref[...]


def _stitch(tc_out, sc_out):
    return pl.pallas_call(
        _stitch_body,
        grid=(B,),
        in_specs=[
            pl.BlockSpec(memory_space=pl.ANY),
            pl.BlockSpec((1, S_SC, D), lambda j: (j, 0, 0)),
        ],
        out_specs=pl.BlockSpec((1, S_SC, D), lambda j: (j, S_TC // S_SC, 0)),
        out_shape=jax.ShapeDtypeStruct((B, S, D), tc_out.dtype),
        input_output_aliases={0: 0},
    )(tc_out, sc_out)


def kernel(x, emb):
    sc_out = _make_sc_add()(x, emb)
    tc_out = _tc_add(x, emb)
    return _stitch(tc_out, sc_out)
